# baseline (speedup 1.0000x reference)
import functools

import jax
import jax.numpy as jnp
from jax import lax
from jax.experimental import pallas as pl
from jax.experimental.pallas import tpu as pltpu
from jax.experimental.pallas import tpu_sc as plsc

BATCH = 4096
HIST = 200
DIM = 64
LABELS = 10

NUM_CORES = 2
NUM_SUBCORES = 16
NUM_WORKERS = NUM_CORES * NUM_SUBCORES
B_PER_W = BATCH // NUM_WORKERS
CHUNK_A = 104
CHUNK_B = HIST - CHUNK_A
NVREG = DIM // 16


def _accum(buf, n, acc):
    def body(r, acc):
        return tuple(acc[d] + buf[r, pl.ds(16 * d, 16)] for d in range(NVREG))
    return lax.fori_loop(0, n, body, acc)


def _sc_mean_body(content_hbm, table_hbm, means_hbm, idx_v, buf_a, buf_b,
                  out_v, sem_a, sem_b):
    c = lax.axis_index("c")
    s = lax.axis_index("s")
    wid = s * NUM_CORES + c
    base = wid * B_PER_W

    pltpu.sync_copy(content_hbm.at[pl.ds(base, B_PER_W)], idx_v)

    pltpu.async_copy(table_hbm.at[idx_v.at[0, pl.ds(0, CHUNK_A)]], buf_a, sem_a)

    def row(b, _):
        pltpu.async_copy(
            table_hbm.at[idx_v.at[b, pl.ds(CHUNK_A, CHUNK_B)]], buf_b, sem_b)
        pltpu.make_async_copy(
            table_hbm.at[idx_v.at[b, pl.ds(0, CHUNK_A)]], buf_a, sem_a).wait()
        acc = tuple(jnp.zeros((16,), jnp.float32) for _ in range(NVREG))
        acc = _accum(buf_a, CHUNK_A, acc)

        @pl.when(b + 1 < B_PER_W)
        def _():
            pltpu.async_copy(
                table_hbm.at[idx_v.at[b + 1, pl.ds(0, CHUNK_A)]], buf_a, sem_a)

        pltpu.make_async_copy(
            table_hbm.at[idx_v.at[b, pl.ds(CHUNK_A, CHUNK_B)]], buf_b,
            sem_b).wait()
        acc = _accum(buf_b, CHUNK_B, acc)
        for d in range(NVREG):
            out_v[b, pl.ds(16 * d, 16)] = acc[d] * (1.0 / HIST)
        return ()

    lax.fori_loop(0, B_PER_W, row, ())
    pltpu.sync_copy(out_v, means_hbm.at[pl.ds(base, B_PER_W)])


_sc_mean = pl.kernel(
    _sc_mean_body,
    out_type=jax.ShapeDtypeStruct((BATCH, DIM), jnp.float32),
    mesh=plsc.VectorSubcoreMesh(core_axis_name="c", subcore_axis_name="s",
                                num_cores=NUM_CORES,
                                num_subcores=NUM_SUBCORES),
    scratch_types=[
        pltpu.VMEM((B_PER_W, HIST), jnp.int32),
        pltpu.VMEM((CHUNK_A, DIM), jnp.float32),
        pltpu.VMEM((CHUNK_B, DIM), jnp.float32),
        pltpu.VMEM((B_PER_W, DIM), jnp.float32),
        pltpu.SemaphoreType.DMA,
        pltpu.SemaphoreType.DMA,
    ],
    compiler_params=pltpu.CompilerParams(use_tc_tiling_on_sc=False),
)


def _fc_body(m_ref, w_ref, b_ref, o_ref):
    o_ref[...] = (
        jnp.dot(m_ref[...], w_ref[...], preferred_element_type=jnp.float32)
        + b_ref[...])


_fc = pl.pallas_call(
    _fc_body,
    out_shape=jax.ShapeDtypeStruct((BATCH, LABELS), jnp.float32),
)


def kernel(content, emb_table, fc_w, fc_b):
    means = _sc_mean(content.astype(jnp.int32), emb_table)
    return _fc(means, fc_w.T, fc_b.reshape(1, LABELS))
